# baseline (speedup 1.0000x reference)
import functools

import jax
import jax.numpy as jnp
import numpy as np
from jax import lax
from jax.experimental import pallas as pl
from jax.experimental.pallas import tpu as pltpu
from jax.experimental.pallas import tpu_sc as plsc

N = 10000
E = 320000
D = 128
NPAD = 10240
NC = 2
NS = 16
NW = NC * NS
EW = E // NW
SUP = 2000
C = 80
NCH = SUP // C
NSUP = EW // SUP
NBUF = 4
RPT = NPAD // NS

_IDX0 = np.zeros((16,), np.int32)

def _deg_body(row_h, col_h, ew_h, degp_h,
              deg_sh, rbuf, cbuf, ewbuf, wm2, ci2, zbuf, sem):
    c = lax.axis_index("c")
    s = lax.axis_index("s")
    wid = c * NS + s

    def zb(i, carry):
        zbuf[pl.ds(i * 16, 16)] = jnp.zeros((16,), jnp.float32)
        return carry
    lax.fori_loop(0, RPT // 16, zb, 0)
    pltpu.sync_copy(zbuf, deg_sh.at[pl.ds(s * RPT, RPT)])
    plsc.subcore_barrier()

    def tbody(t, carry):
        base = wid * EW + t * SUP
        pltpu.sync_copy(row_h.at[pl.ds(base, SUP)], rbuf)
        pltpu.sync_copy(col_h.at[pl.ds(base, SUP)], cbuf)
        pltpu.sync_copy(ew_h.at[pl.ds(base, SUP)], ewbuf)
        descs = []
        for j in range(NCH):
            for k in range(C // 16):
                sl = pl.ds(j * C + k * 16, 16)
                r = rbuf[sl]
                cc = cbuf[sl]
                w = ewbuf[sl]
                wm2[j, pl.ds(k * 16, 16)] = jnp.where(r != cc, w, 0.0)
                ci2[j, pl.ds(k * 16, 16)] = cc
            descs.append(
                pltpu.async_copy(wm2.at[j], deg_sh.at[ci2.at[j]], sem, add=True))
        for dsc in descs:
            dsc.wait()
        return carry
    lax.fori_loop(0, NSUP, tbody, 0)

    plsc.subcore_barrier()
    pltpu.sync_copy(deg_sh.at[pl.ds(s * RPT, RPT)],
                    degp_h.at[c, pl.ds(s * RPT, RPT)])


def _spmm_body(hp_h, row_h, col_h, ew_h, out_h,
               acc_sh, rbig, cbig, wmbuf, *bufs):
    rows = list(bufs[:NBUF])
    cid = list(bufs[NBUF:2 * NBUF])
    gsem = list(bufs[2 * NBUF:3 * NBUF])
    ssem = list(bufs[3 * NBUF:4 * NBUF])
    c = lax.axis_index("c")
    s = lax.axis_index("s")
    wid = c * NS + s

    zrows = rows[NBUF - 1]
    def zb(i, carry):
        for k in range(D // 16):
            zrows[i, pl.ds(k * 16, 16)] = jnp.zeros((16,), jnp.float32)
        return carry
    lax.fori_loop(0, C, zb, 0)
    for k in range(RPT // C):
        pltpu.sync_copy(zrows, acc_sh.at[pl.ds(s * RPT + k * C, C), :])
    plsc.subcore_barrier()

    def fill_cid(bn, jn):
        for k in range(C // 16):
            cid[bn][pl.ds(k * 16, 16)] = cbig[pl.ds(jn * C + k * 16, 16)]

    def gather(bn, jn):
        return pltpu.async_copy(
            hp_h.at[rbig.at[pl.ds(jn * C, C)]], rows[bn], gsem[bn])

    def tbody(t, carry):
        base = wid * EW + t * SUP
        pltpu.sync_copy(row_h.at[pl.ds(base, SUP)], rbig)
        pltpu.sync_copy(col_h.at[pl.ds(base, SUP)], cbig)
        pltpu.sync_copy(ew_h.at[pl.ds(base, SUP)], wmbuf.at[pl.ds(0, SUP)])
        gdesc = [None] * NBUF
        sdesc = [None] * NBUF
        for b in range(NBUF - 1):
            fill_cid(b, b)
            gdesc[b] = gather(b, b)

        def mbody(i, carry):
            sl = pl.ds(i * 16, 16)
            wmbuf[sl] = jnp.where(rbig[sl] != cbig[sl], wmbuf[sl], 0.0)
            return carry
        lax.fori_loop(0, SUP // 16, mbody, 0)

        for j in range(NCH):
            b = j % NBUF
            gdesc[b].wait()
            rb = rows[b]
            off = j * C

            def sbody(g, carry):
                for u in range(4):
                    e = g * 4 + u
                    wv = wmbuf[pl.ds(off + e, 16)]
                    w = wv[0]
                    for k in range(D // 16):
                        sl = pl.ds(k * 16, 16)
                        rb[e, sl] = rb[e, sl] * w
                return carry
            lax.fori_loop(0, C // 4, sbody, 0)
            bp = (j - 1) % NBUF
            if j > 0 and sdesc[bp] is not None:
                sdesc[bp].wait()
                sdesc[bp] = None
            sdesc[b] = pltpu.async_copy(
                rb, acc_sh.at[cid[b]], ssem[b], add=True)
            jn = j + NBUF - 1
            if jn < NCH:
                bn = jn % NBUF
                if sdesc[bn] is not None:
                    sdesc[bn].wait()
                    sdesc[bn] = None
                fill_cid(bn, jn)
                gdesc[bn] = gather(bn, jn)
        for b in range(NBUF):
            if sdesc[b] is not None:
                sdesc[b].wait()
        return carry
    lax.fori_loop(0, NSUP, tbody, 0)

    plsc.subcore_barrier()
    pltpu.sync_copy(acc_sh.at[pl.ds(s * RPT, RPT), :],
                    out_h.at[c, pl.ds(s * RPT, RPT), :])


@functools.lru_cache(maxsize=1)
def _sc_kernels():
    mesh = plsc.VectorSubcoreMesh(
        core_axis_name="c", subcore_axis_name="s",
        num_cores=NC, num_subcores=NS)
    deg_call = pl.kernel(
        _deg_body,
        out_type=jax.ShapeDtypeStruct((NC, NPAD), jnp.float32),
        mesh=mesh,
        scratch_types=[
            pltpu.VMEM_SHARED((NPAD,), jnp.float32),
            pltpu.VMEM((SUP,), jnp.int32),
            pltpu.VMEM((SUP,), jnp.int32),
            pltpu.VMEM((SUP,), jnp.float32),
            pltpu.VMEM((NCH, C), jnp.float32),
            pltpu.VMEM((NCH, C), jnp.int32),
            pltpu.VMEM((RPT,), jnp.float32),
            pltpu.SemaphoreType.DMA,
        ],
    )
    spmm_call = pl.kernel(
        _spmm_body,
        out_type=jax.ShapeDtypeStruct((NC, NPAD, D), jnp.float32),
        mesh=mesh,
        scratch_types=(
            [
                pltpu.VMEM_SHARED((NPAD, D), jnp.float32),
                pltpu.VMEM((SUP,), jnp.int32),
                pltpu.VMEM((SUP,), jnp.int32),
                pltpu.VMEM((SUP + 16,), jnp.float32),
            ]
            + [pltpu.VMEM((C, D), jnp.float32) for _ in range(NBUF)]
            + [pltpu.VMEM((C,), jnp.int32) for _ in range(NBUF)]
            + [pltpu.SemaphoreType.DMA for _ in range(2 * NBUF)]
        ),
    )
    return deg_call, spmm_call


BLK = 1000
GRID = N // BLK


def _k0_body(emb_ref, w1_ref, h1_ref):
    h1_ref[...] = jnp.dot(emb_ref[...], w1_ref[...],
                          preferred_element_type=jnp.float32)


def _k0(emb, W1):
    return pl.pallas_call(
        _k0_body,
        grid=(GRID,),
        in_specs=[
            pl.BlockSpec((BLK, D), lambda i: (i, 0)),
            pl.BlockSpec((D, D), lambda i: (0, 0)),
        ],
        out_specs=pl.BlockSpec((BLK, D), lambda i: (i, 0)),
        out_shape=jax.ShapeDtypeStruct((N, D), jnp.float32),
    )(emb, W1)


def _k1_body(h1_ref, d0_ref, d1_ref, h1p_ref, dis_ref):
    dis = lax.rsqrt(d0_ref[...] + d1_ref[...] + 1.0)
    h1p_ref[...] = h1_ref[...] * dis
    dis_ref[...] = dis


def _k1(h1, d0, d1):
    return pl.pallas_call(
        _k1_body,
        grid=(GRID,),
        in_specs=[
            pl.BlockSpec((BLK, D), lambda i: (i, 0)),
            pl.BlockSpec((BLK, 1), lambda i: (i, 0)),
            pl.BlockSpec((BLK, 1), lambda i: (i, 0)),
        ],
        out_specs=[
            pl.BlockSpec((BLK, D), lambda i: (i, 0)),
            pl.BlockSpec((BLK, 1), lambda i: (i, 0)),
        ],
        out_shape=[
            jax.ShapeDtypeStruct((N, D), jnp.float32),
            jax.ShapeDtypeStruct((N, 1), jnp.float32),
        ],
    )(h1, d0, d1)


def _k2_body(sp0_ref, sp1_ref, h1p_ref, dis_ref, b1_ref, w2_ref,
             z1_ref, h2p_ref):
    dis = dis_ref[...]
    a = (sp0_ref[0] + sp1_ref[0] + h1p_ref[...]) * dis + b1_ref[...]
    z1 = jnp.maximum(a, 0.0)
    z1_ref[...] = z1
    h2p_ref[...] = jnp.dot(z1, w2_ref[...],
                           preferred_element_type=jnp.float32) * dis


def _k2(Sp, h1p, dis, b1, W2):
    return pl.pallas_call(
        _k2_body,
        grid=(GRID,),
        in_specs=[
            pl.BlockSpec((1, BLK, D), lambda i: (0, i, 0)),
            pl.BlockSpec((1, BLK, D), lambda i: (1, i, 0)),
            pl.BlockSpec((BLK, D), lambda i: (i, 0)),
            pl.BlockSpec((BLK, 1), lambda i: (i, 0)),
            pl.BlockSpec((1, D), lambda i: (0, 0)),
            pl.BlockSpec((D, D), lambda i: (0, 0)),
        ],
        out_specs=[
            pl.BlockSpec((BLK, D), lambda i: (i, 0)),
            pl.BlockSpec((BLK, D), lambda i: (i, 0)),
        ],
        out_shape=[
            jax.ShapeDtypeStruct((N, D), jnp.float32),
            jax.ShapeDtypeStruct((N, D), jnp.float32),
        ],
    )(Sp, Sp, h1p, dis, b1, W2)


def _k3_body(sp0_ref, sp1_ref, h2p_ref, dis_ref, b2_ref, emb_ref, z1_ref,
             out_ref):
    a = (sp0_ref[0] + sp1_ref[0] + h2p_ref[...]) * dis_ref[...] + b2_ref[...]
    out_ref[:, 0:D] = emb_ref[...]
    out_ref[:, D:2 * D] = z1_ref[...]
    out_ref[:, 2 * D:3 * D] = jnp.maximum(a, 0.0)


def _k3(Sp, h2p, dis, b2, emb, z1):
    return pl.pallas_call(
        _k3_body,
        grid=(GRID,),
        in_specs=[
            pl.BlockSpec((1, BLK, D), lambda i: (0, i, 0)),
            pl.BlockSpec((1, BLK, D), lambda i: (1, i, 0)),
            pl.BlockSpec((BLK, D), lambda i: (i, 0)),
            pl.BlockSpec((BLK, 1), lambda i: (i, 0)),
            pl.BlockSpec((1, D), lambda i: (0, 0)),
            pl.BlockSpec((BLK, D), lambda i: (i, 0)),
            pl.BlockSpec((BLK, D), lambda i: (i, 0)),
        ],
        out_specs=pl.BlockSpec((BLK, 3 * D), lambda i: (i, 0)),
        out_shape=jax.ShapeDtypeStruct((N, 3 * D), jnp.float32),
    )(Sp, Sp, h2p, dis, b2, emb, z1)


def kernel(x, homo_edge_index, edge_weight, embedding, W1, b1, W2, b2):
    del x
    row = homo_edge_index[0]
    col = homo_edge_index[1]
    _deg_call, _spmm_call = _sc_kernels()
    h1 = _k0(embedding, W1)
    degp = _deg_call(row, col, edge_weight)
    d0 = degp[0, :N].reshape(N, 1)
    d1 = degp[1, :N].reshape(N, 1)
    h1p, dis = _k1(h1, d0, d1)
    Sp1 = _spmm_call(h1p, row, col, edge_weight)
    z1, h2p = _k2(Sp1, h1p, dis, b1.reshape(1, D), W2)
    Sp2 = _spmm_call(h2p, row, col, edge_weight)
    out = _k3(Sp2, h2p, dis, b2.reshape(1, D), embedding, z1)
    return out


# speedup vs baseline: 1.0045x; 1.0045x over previous
import functools

import jax
import jax.numpy as jnp
import numpy as np
from jax import lax
from jax.experimental import pallas as pl
from jax.experimental.pallas import tpu as pltpu
from jax.experimental.pallas import tpu_sc as plsc

N = 10000
E = 320000
D = 128
NPAD = 10240
NC = 2
NS = 16
NW = NC * NS
EW = E // NW
SUP = 2000
C = 80
NCH = SUP // C
NSUP = EW // SUP
NBUF = 4
RPT = NPAD // NS

_IDX0 = np.zeros((16,), np.int32)

def _deg_body(row_h, col_h, ew_h, degp_h,
              deg_sh, rbuf, cbuf, ewbuf, wm2, ci2, zbuf, sem):
    c = lax.axis_index("c")
    s = lax.axis_index("s")
    wid = c * NS + s

    def zb(i, carry):
        zbuf[pl.ds(i * 16, 16)] = jnp.zeros((16,), jnp.float32)
        return carry
    lax.fori_loop(0, RPT // 16, zb, 0)
    pltpu.sync_copy(zbuf, deg_sh.at[pl.ds(s * RPT, RPT)])
    plsc.subcore_barrier()

    def tbody(t, carry):
        base = wid * EW + t * SUP
        pltpu.sync_copy(row_h.at[pl.ds(base, SUP)], rbuf)
        pltpu.sync_copy(col_h.at[pl.ds(base, SUP)], cbuf)
        pltpu.sync_copy(ew_h.at[pl.ds(base, SUP)], ewbuf)
        descs = []
        for j in range(NCH):
            for k in range(C // 16):
                sl = pl.ds(j * C + k * 16, 16)
                r = rbuf[sl]
                cc = cbuf[sl]
                w = ewbuf[sl]
                wm2[j, pl.ds(k * 16, 16)] = jnp.where(r != cc, w, 0.0)
                ci2[j, pl.ds(k * 16, 16)] = cc
            descs.append(
                pltpu.async_copy(wm2.at[j], deg_sh.at[ci2.at[j]], sem, add=True))
        for dsc in descs:
            dsc.wait()
        return carry
    lax.fori_loop(0, NSUP, tbody, 0)

    plsc.subcore_barrier()
    pltpu.sync_copy(deg_sh.at[pl.ds(s * RPT, RPT)],
                    degp_h.at[c, pl.ds(s * RPT, RPT)])


def _spmm_body(hp_h, row_h, col_h, ew_h, out_h,
               acc_sh, rbig, cbig, wmbuf, *bufs):
    rows = list(bufs[:NBUF])
    cid = list(bufs[NBUF:2 * NBUF])
    gsem = list(bufs[2 * NBUF:3 * NBUF])
    ssem = list(bufs[3 * NBUF:4 * NBUF])
    c = lax.axis_index("c")
    s = lax.axis_index("s")
    wid = c * NS + s

    zrows = rows[NBUF - 1]
    def zb(i, carry):
        for k in range(D // 16):
            zrows[i, pl.ds(k * 16, 16)] = jnp.zeros((16,), jnp.float32)
        return carry
    lax.fori_loop(0, C, zb, 0)
    for k in range(RPT // C):
        pltpu.sync_copy(zrows, acc_sh.at[pl.ds(s * RPT + k * C, C), :])
    plsc.subcore_barrier()

    def fill_cid(bn, jn):
        for k in range(C // 16):
            cid[bn][pl.ds(k * 16, 16)] = cbig[pl.ds(jn * C + k * 16, 16)]

    def gather(bn, jn):
        return pltpu.async_copy(
            hp_h.at[rbig.at[pl.ds(jn * C, C)]], rows[bn], gsem[bn])

    def tbody(t, carry):
        base = wid * EW + t * SUP
        pltpu.sync_copy(row_h.at[pl.ds(base, SUP)], rbig)
        pltpu.sync_copy(col_h.at[pl.ds(base, SUP)], cbig)
        pltpu.sync_copy(ew_h.at[pl.ds(base, SUP)], wmbuf.at[pl.ds(0, SUP)])
        gdesc = [None] * NBUF
        sdesc = [None] * NBUF
        for b in range(NBUF - 1):
            fill_cid(b, b)
            gdesc[b] = gather(b, b)

        def mbody(i, carry):
            sl = pl.ds(i * 16, 16)
            wmbuf[sl] = jnp.where(rbig[sl] != cbig[sl], wmbuf[sl], 0.0)
            return carry
        lax.fori_loop(0, SUP // 16, mbody, 0)

        for j in range(NCH):
            b = j % NBUF
            gdesc[b].wait()
            rb = rows[b]
            off = j * C

            def sbody(g, carry):
                for u in range(4):
                    e = g * 4 + u
                    wv = wmbuf[pl.ds(off + e, 16)]
                    w = wv[0]
                    for k in range(D // 16):
                        sl = pl.ds(k * 16, 16)
                        rb[e, sl] = rb[e, sl] * w
                return carry
            lax.fori_loop(0, C // 4, sbody, 0)
            bp = (j - 1) % NBUF
            if j > 0 and sdesc[bp] is not None:
                sdesc[bp].wait()
                sdesc[bp] = None
            sdesc[b] = pltpu.async_copy(
                rb, acc_sh.at[cid[b]], ssem[b], add=True)
            jn = j + NBUF - 1
            if jn < NCH:
                bn = jn % NBUF
                if sdesc[bn] is not None:
                    sdesc[bn].wait()
                    sdesc[bn] = None
                fill_cid(bn, jn)
                gdesc[bn] = gather(bn, jn)
        for b in range(NBUF):
            if sdesc[b] is not None:
                sdesc[b].wait()
        return carry
    lax.fori_loop(0, NSUP, tbody, 0)

    plsc.subcore_barrier()
    pltpu.sync_copy(acc_sh.at[pl.ds(s * RPT, RPT), :],
                    out_h.at[c, pl.ds(s * RPT, RPT), :])


@functools.lru_cache(maxsize=1)
def _sc_kernels():
    mesh = plsc.VectorSubcoreMesh(
        core_axis_name="c", subcore_axis_name="s",
        num_cores=NC, num_subcores=NS)
    deg_call = pl.kernel(
        _deg_body,
        out_type=jax.ShapeDtypeStruct((NC, NPAD), jnp.float32),
        mesh=mesh,
        scratch_types=[
            pltpu.VMEM_SHARED((NPAD,), jnp.float32),
            pltpu.VMEM((SUP,), jnp.int32),
            pltpu.VMEM((SUP,), jnp.int32),
            pltpu.VMEM((SUP,), jnp.float32),
            pltpu.VMEM((NCH, C), jnp.float32),
            pltpu.VMEM((NCH, C), jnp.int32),
            pltpu.VMEM((RPT,), jnp.float32),
            pltpu.SemaphoreType.DMA,
        ],
    )
    spmm_call = pl.kernel(
        _spmm_body,
        out_type=jax.ShapeDtypeStruct((NC, NPAD, D), jnp.float32),
        mesh=mesh,
        scratch_types=(
            [
                pltpu.VMEM_SHARED((NPAD, D), jnp.float32),
                pltpu.VMEM((SUP,), jnp.int32),
                pltpu.VMEM((SUP,), jnp.int32),
                pltpu.VMEM((SUP + 16,), jnp.float32),
            ]
            + [pltpu.VMEM((C, D), jnp.float32) for _ in range(NBUF)]
            + [pltpu.VMEM((C,), jnp.int32) for _ in range(NBUF)]
            + [pltpu.SemaphoreType.DMA for _ in range(2 * NBUF)]
        ),
    )
    return deg_call, spmm_call


BLK = 1000
GRID = N // BLK


def _k1_body(emb_ref, w1_ref, d0_ref, d1_ref, h1p_ref, dis_ref):
    dis = lax.rsqrt(d0_ref[...] + d1_ref[...] + 1.0)
    mm = jnp.dot(emb_ref[...], w1_ref[...],
                 preferred_element_type=jnp.float32)
    h1p_ref[...] = mm * dis
    dis_ref[...] = dis


def _k1(emb, W1, d0, d1):
    return pl.pallas_call(
        _k1_body,
        grid=(GRID,),
        in_specs=[
            pl.BlockSpec((BLK, D), lambda i: (i, 0)),
            pl.BlockSpec((D, D), lambda i: (0, 0)),
            pl.BlockSpec((BLK, 1), lambda i: (i, 0)),
            pl.BlockSpec((BLK, 1), lambda i: (i, 0)),
        ],
        out_specs=[
            pl.BlockSpec((BLK, D), lambda i: (i, 0)),
            pl.BlockSpec((BLK, 1), lambda i: (i, 0)),
        ],
        out_shape=[
            jax.ShapeDtypeStruct((N, D), jnp.float32),
            jax.ShapeDtypeStruct((N, 1), jnp.float32),
        ],
    )(emb, W1, d0, d1)


def _k2_body(sp0_ref, sp1_ref, h1p_ref, dis_ref, b1_ref, w2_ref,
             z1_ref, h2p_ref):
    dis = dis_ref[...]
    a = (sp0_ref[0] + sp1_ref[0] + h1p_ref[...]) * dis + b1_ref[...]
    z1 = jnp.maximum(a, 0.0)
    z1_ref[...] = z1
    h2p_ref[...] = jnp.dot(z1, w2_ref[...],
                           preferred_element_type=jnp.float32) * dis


def _k2(Sp, h1p, dis, b1, W2):
    return pl.pallas_call(
        _k2_body,
        grid=(GRID,),
        in_specs=[
            pl.BlockSpec((1, BLK, D), lambda i: (0, i, 0)),
            pl.BlockSpec((1, BLK, D), lambda i: (1, i, 0)),
            pl.BlockSpec((BLK, D), lambda i: (i, 0)),
            pl.BlockSpec((BLK, 1), lambda i: (i, 0)),
            pl.BlockSpec((1, D), lambda i: (0, 0)),
            pl.BlockSpec((D, D), lambda i: (0, 0)),
        ],
        out_specs=[
            pl.BlockSpec((BLK, D), lambda i: (i, 0)),
            pl.BlockSpec((BLK, D), lambda i: (i, 0)),
        ],
        out_shape=[
            jax.ShapeDtypeStruct((N, D), jnp.float32),
            jax.ShapeDtypeStruct((N, D), jnp.float32),
        ],
    )(Sp, Sp, h1p, dis, b1, W2)


def _k3_body(sp0_ref, sp1_ref, h2p_ref, dis_ref, b2_ref, emb_ref, z1_ref,
             out_ref):
    a = (sp0_ref[0] + sp1_ref[0] + h2p_ref[...]) * dis_ref[...] + b2_ref[...]
    out_ref[:, 0:D] = emb_ref[...]
    out_ref[:, D:2 * D] = z1_ref[...]
    out_ref[:, 2 * D:3 * D] = jnp.maximum(a, 0.0)


def _k3(Sp, h2p, dis, b2, emb, z1):
    return pl.pallas_call(
        _k3_body,
        grid=(GRID,),
        in_specs=[
            pl.BlockSpec((1, BLK, D), lambda i: (0, i, 0)),
            pl.BlockSpec((1, BLK, D), lambda i: (1, i, 0)),
            pl.BlockSpec((BLK, D), lambda i: (i, 0)),
            pl.BlockSpec((BLK, 1), lambda i: (i, 0)),
            pl.BlockSpec((1, D), lambda i: (0, 0)),
            pl.BlockSpec((BLK, D), lambda i: (i, 0)),
            pl.BlockSpec((BLK, D), lambda i: (i, 0)),
        ],
        out_specs=pl.BlockSpec((BLK, 3 * D), lambda i: (i, 0)),
        out_shape=jax.ShapeDtypeStruct((N, 3 * D), jnp.float32),
    )(Sp, Sp, h2p, dis, b2, emb, z1)


def kernel(x, homo_edge_index, edge_weight, embedding, W1, b1, W2, b2):
    del x
    row = homo_edge_index[0]
    col = homo_edge_index[1]
    _deg_call, _spmm_call = _sc_kernels()
    degp = _deg_call(row, col, edge_weight)
    d0 = degp[0, :N].reshape(N, 1)
    d1 = degp[1, :N].reshape(N, 1)
    h1p, dis = _k1(embedding, W1, d0, d1)
    Sp1 = _spmm_call(h1p, row, col, edge_weight)
    z1, h2p = _k2(Sp1, h1p, dis, b1.reshape(1, D), W2)
    Sp2 = _spmm_call(h2p, row, col, edge_weight)
    out = _k3(Sp2, h2p, dis, b2.reshape(1, D), embedding, z1)
    return out
